# baseline (speedup 1.0000x reference)
import jax
import jax.numpy as jnp
from jax.experimental import pallas as pl
from jax.experimental.pallas import tpu as pltpu

_VMEM_LIMIT = 48 * 1024 * 1024
_ROW_TILE = 256


def _gru_cell(w0, wih, whh, bih, bhh):
    i_r = jnp.dot(w0, wih[0], preferred_element_type=jnp.float32) + bih[0]
    i_z = jnp.dot(w0, wih[1], preferred_element_type=jnp.float32) + bih[1]
    i_n = jnp.dot(w0, wih[2], preferred_element_type=jnp.float32) + bih[2]
    h_r = jnp.dot(w0, whh[0], preferred_element_type=jnp.float32) + bhh[0]
    h_z = jnp.dot(w0, whh[1], preferred_element_type=jnp.float32) + bhh[1]
    h_n = jnp.dot(w0, whh[2], preferred_element_type=jnp.float32) + bhh[2]
    r = jax.nn.sigmoid(i_r + h_r)
    z = jax.nn.sigmoid(i_z + h_z)
    n = jnp.tanh(i_n + r * h_n)
    return (1.0 - z) * n + z * w0


def _prep_kernel(x_ref, w01_ref, wih1_ref, whh1_ref, bih1_ref, bhh1_ref,
                 w02_ref, wih2_ref, whh2_ref, bih2_ref, bhh2_ref,
                 v1_ref, w2e_ref):
    w1e = _gru_cell(w01_ref[...], wih1_ref[...], whh1_ref[...],
                    bih1_ref[...], bhh1_ref[...])
    w2e = _gru_cell(w02_ref[...], wih2_ref[...], whh2_ref[...],
                    bih2_ref[...], bhh2_ref[...])
    w2e_ref[...] = w2e
    v1_ref[...] = jnp.dot(x_ref[...], w1e, preferred_element_type=jnp.float32)


def _layer_kernel(a_ref, v_ref, o_ref):
    a = a_ref[...]
    v = v_ref[...].astype(a.dtype)
    o_ref[...] = jnp.tanh(
        jnp.dot(a, v, preferred_element_type=jnp.float32))


def _xw_kernel(h_ref, w_ref, v_ref):
    v_ref[...] = jnp.dot(h_ref[...], w_ref[...],
                         preferred_element_type=jnp.float32)


def _layer(a_norm, v, n, d):
    tm = _ROW_TILE
    return pl.pallas_call(
        _layer_kernel,
        out_shape=jax.ShapeDtypeStruct((n, d), jnp.float32),
        grid=(n // tm,),
        in_specs=[
            pl.BlockSpec((tm, n), lambda i: (i, 0)),
            pl.BlockSpec((n, d), lambda i: (0, 0)),
        ],
        out_specs=pl.BlockSpec((tm, d), lambda i: (i, 0)),
        compiler_params=pltpu.CompilerParams(
            dimension_semantics=("parallel",),
            vmem_limit_bytes=_VMEM_LIMIT,
        ),
    )(a_norm, v)


def kernel(conv1_initial_weight, conv1_w_ih, conv1_w_hh, conv1_b_ih, conv1_b_hh,
           conv2_initial_weight, conv2_w_ih, conv2_w_hh, conv2_b_ih, conv2_b_hh,
           users, x, edge_index):
    n, d = x.shape
    src = edge_index[0]
    dst = edge_index[1]

    ones = jnp.ones(src.shape, jnp.float32)
    incount = jnp.zeros((n,), jnp.float32).at[dst].add(ones)
    selfw = jnp.zeros((n,), jnp.float32).at[dst].add(
        jnp.where(src == dst, 1.0, 0.0))
    deg = incount + jnp.where(selfw == 0.0, 1.0, 0.0)
    dis = jnp.where(deg > 0.0, jax.lax.rsqrt(deg), 0.0)
    vals = (dis[dst] * dis[src]).astype(jnp.bfloat16)
    a = jnp.zeros((n, n), jnp.bfloat16).at[dst, src].add(vals)
    idx = jnp.arange(n)
    diag_fill = jnp.where(selfw == 0.0, dis * dis, 0.0).astype(jnp.bfloat16)
    a = a.at[idx, idx].add(diag_fill)

    v1, w2e = pl.pallas_call(
        _prep_kernel,
        out_shape=(jax.ShapeDtypeStruct((n, d), jnp.float32),
                   jax.ShapeDtypeStruct((d, d), jnp.float32)),
        compiler_params=pltpu.CompilerParams(
            vmem_limit_bytes=_VMEM_LIMIT,
        ),
    )(x, conv1_initial_weight, conv1_w_ih, conv1_w_hh, conv1_b_ih, conv1_b_hh,
      conv2_initial_weight, conv2_w_ih, conv2_w_hh, conv2_b_ih, conv2_b_hh)

    h = _layer(a, v1, n, d)

    v2 = pl.pallas_call(
        _xw_kernel,
        out_shape=jax.ShapeDtypeStruct((n, d), jnp.float32),
        compiler_params=pltpu.CompilerParams(
            vmem_limit_bytes=_VMEM_LIMIT,
        ),
    )(h, w2e)

    out = _layer(a, v2, n, d)
    return out[users]


# speedup vs baseline: 8.9300x; 8.9300x over previous
import jax
import jax.numpy as jnp
from jax.experimental import pallas as pl
from jax.experimental.pallas import tpu as pltpu

_VMEM_LIMIT = 48 * 1024 * 1024
_ROW_TILE = 128


def _gru_cell(w0, wih, whh, bih, bhh):
    i_r = jnp.dot(w0, wih[0], preferred_element_type=jnp.float32) + bih[0]
    i_z = jnp.dot(w0, wih[1], preferred_element_type=jnp.float32) + bih[1]
    i_n = jnp.dot(w0, wih[2], preferred_element_type=jnp.float32) + bih[2]
    h_r = jnp.dot(w0, whh[0], preferred_element_type=jnp.float32) + bhh[0]
    h_z = jnp.dot(w0, whh[1], preferred_element_type=jnp.float32) + bhh[1]
    h_n = jnp.dot(w0, whh[2], preferred_element_type=jnp.float32) + bhh[2]
    r = jax.nn.sigmoid(i_r + h_r)
    z = jax.nn.sigmoid(i_z + h_z)
    n = jnp.tanh(i_n + r * h_n)
    return (1.0 - z) * n + z * w0


def _rowsum_kernel(a_ref, o_ref):
    o_ref[...] = jnp.sum(a_ref[...], axis=1, keepdims=True)


def _prep_kernel(x_ref, dis_ref,
                 w01_ref, wih1_ref, whh1_ref, bih1_ref, bhh1_ref,
                 w02_ref, wih2_ref, whh2_ref, bih2_ref, bhh2_ref,
                 dv1_ref, w2e_ref):
    w1e = _gru_cell(w01_ref[...], wih1_ref[...], whh1_ref[...],
                    bih1_ref[...], bhh1_ref[...])
    w2e = _gru_cell(w02_ref[...], wih2_ref[...], whh2_ref[...],
                    bih2_ref[...], bhh2_ref[...])
    w2e_ref[...] = w2e
    dv1_ref[...] = dis_ref[...] * jnp.dot(
        x_ref[...], w1e, preferred_element_type=jnp.float32)


def _xw_kernel(h_ref, w_ref, dis_ref, dv_ref):
    dv_ref[...] = dis_ref[...] * jnp.dot(
        h_ref[...], w_ref[...], preferred_element_type=jnp.float32)


def _layer_kernel(a_ref, dv_ref, dis_ref, fill_ref, dvblk_ref, o_ref):
    acc = jnp.dot(a_ref[...], dv_ref[...], preferred_element_type=jnp.float32)
    acc = acc + fill_ref[...] * dvblk_ref[...]
    o_ref[...] = jnp.tanh(dis_ref[...] * acc)


def _layer(a_raw, dv, dis2d, fill2d, n, d):
    tm = _ROW_TILE
    return pl.pallas_call(
        _layer_kernel,
        out_shape=jax.ShapeDtypeStruct((n, d), jnp.float32),
        grid=(n // tm,),
        in_specs=[
            pl.BlockSpec((tm, n), lambda i: (i, 0)),
            pl.BlockSpec((n, d), lambda i: (0, 0)),
            pl.BlockSpec((tm, 1), lambda i: (i, 0)),
            pl.BlockSpec((tm, 1), lambda i: (i, 0)),
            pl.BlockSpec((tm, d), lambda i: (i, 0)),
        ],
        out_specs=pl.BlockSpec((tm, d), lambda i: (i, 0)),
        compiler_params=pltpu.CompilerParams(
            dimension_semantics=("parallel",),
            vmem_limit_bytes=_VMEM_LIMIT,
        ),
    )(a_raw, dv, dis2d, fill2d, dv)


def kernel(conv1_initial_weight, conv1_w_ih, conv1_w_hh, conv1_b_ih, conv1_b_hh,
           conv2_initial_weight, conv2_w_ih, conv2_w_hh, conv2_b_ih, conv2_b_hh,
           users, x, edge_index):
    n, d = x.shape
    src = edge_index[0]
    dst = edge_index[1]
    tm = _ROW_TILE

    a_raw = jnp.zeros((n, n), jnp.float32).at[dst, src].add(
        jnp.ones(src.shape, jnp.float32))
    idx = jnp.arange(n)
    diag = a_raw[idx, idx]

    rs = pl.pallas_call(
        _rowsum_kernel,
        out_shape=jax.ShapeDtypeStruct((n, 1), jnp.float32),
        grid=(n // tm,),
        in_specs=[pl.BlockSpec((tm, n), lambda i: (i, 0))],
        out_specs=pl.BlockSpec((tm, 1), lambda i: (i, 0)),
        compiler_params=pltpu.CompilerParams(
            dimension_semantics=("parallel",),
            vmem_limit_bytes=_VMEM_LIMIT,
        ),
    )(a_raw)

    fill = jnp.where(diag == 0.0, 1.0, 0.0)
    deg = rs[:, 0] + fill
    dis = jnp.where(deg > 0.0, jax.lax.rsqrt(deg), 0.0)
    dis2d = dis[:, None]
    fill2d = fill[:, None]

    dv1, w2e = pl.pallas_call(
        _prep_kernel,
        out_shape=(jax.ShapeDtypeStruct((n, d), jnp.float32),
                   jax.ShapeDtypeStruct((d, d), jnp.float32)),
        compiler_params=pltpu.CompilerParams(
            vmem_limit_bytes=_VMEM_LIMIT,
        ),
    )(x, dis2d, conv1_initial_weight, conv1_w_ih, conv1_w_hh, conv1_b_ih,
      conv1_b_hh, conv2_initial_weight, conv2_w_ih, conv2_w_hh, conv2_b_ih,
      conv2_b_hh)

    h = _layer(a_raw, dv1, dis2d, fill2d, n, d)

    dv2 = pl.pallas_call(
        _xw_kernel,
        out_shape=jax.ShapeDtypeStruct((n, d), jnp.float32),
        compiler_params=pltpu.CompilerParams(
            vmem_limit_bytes=_VMEM_LIMIT,
        ),
    )(h, w2e, dis2d)

    out = _layer(a_raw, dv2, dis2d, fill2d, n, d)
    return out[users]
